# Initial kernel scaffold; baseline (speedup 1.0000x reference)
import jax
import jax.numpy as jnp
from jax.experimental import pallas as pl


def kernel(predicted_positions, real_positions, real_expressions):
    raise NotImplementedError("write your pallas kernel here")


# speedup vs baseline: 3.1049x; 3.1049x over previous
import functools

import jax
import jax.numpy as jnp
from jax import lax
from jax.experimental import pallas as pl
from jax.experimental.pallas import tpu as pltpu
from jax.experimental.pallas import tpu_sc as plsc

_QB = 512
_BIG = 2 ** 30


def _argmin_body(n_keys, q_ref, qs_ref, ks_ref, kt_ref, out_ref):
    q = q_ref[0]
    a2 = jnp.sum(q * q, axis=1, keepdims=True)
    k = kt_ref[0]
    b2 = jnp.sum(k * k, axis=0, keepdims=True)
    mm = jnp.dot(qs_ref[0], ks_ref[0], preferred_element_type=jnp.float32)
    d2 = (a2 + b2) + mm
    m = jnp.min(d2, axis=1, keepdims=True)
    mc = jnp.maximum(m, 0.0)
    iota = lax.broadcasted_iota(jnp.int32, d2.shape, 1)
    idx = jnp.min(jnp.where(d2 <= mc, iota, _BIG), axis=1)
    out_ref[0, 0, :] = idx + pl.program_id(0) * n_keys


def _nearest_indices(pred, qs, ks, kt):
    B, N, _ = pred.shape
    grid = (B, N // _QB)
    return pl.pallas_call(
        functools.partial(_argmin_body, N),
        grid=grid,
        in_specs=[
            pl.BlockSpec((1, _QB, 2), lambda b, i: (b, i, 0)),
            pl.BlockSpec((1, _QB, 2), lambda b, i: (b, i, 0)),
            pl.BlockSpec((1, 2, N), lambda b, i: (b, 0, 0)),
            pl.BlockSpec((1, 2, N), lambda b, i: (b, 0, 0)),
        ],
        out_specs=pl.BlockSpec((1, 1, _QB), lambda b, i: (b * (N // _QB) + i, 0, 0)),
        out_shape=jax.ShapeDtypeStruct((B * (N // _QB), 1, _QB), jnp.int32),
    )(pred, qs, ks, kt)


_NC = 2
_NS = 16
_NW = _NC * _NS
_CH = 128


def _gather_rows_sc(table, idx):
    rows, G = table.shape
    per_w = rows // _NW
    nch = per_w // _CH
    mesh = plsc.VectorSubcoreMesh(core_axis_name="c", subcore_axis_name="s")

    @functools.partial(
        pl.kernel, mesh=mesh,
        out_type=jax.ShapeDtypeStruct((rows, G), jnp.float32),
        scratch_types=[
            pltpu.VMEM((per_w,), jnp.int32),
            pltpu.VMEM((_CH, G), jnp.float32),
            pltpu.VMEM((_CH, G), jnp.float32),
            pltpu.SemaphoreType.DMA,
            pltpu.SemaphoreType.DMA,
        ],
    )
    def k(table_hbm, idx_hbm, out_hbm, idx_v, buf0, buf1, sem0, sem1):
        wid = lax.axis_index("s") * _NC + lax.axis_index("c")
        base = wid * per_w
        pltpu.sync_copy(idx_hbm.at[pl.ds(base, per_w)], idx_v)
        bufs = (buf0, buf1)
        sems = (sem0, sem1)
        cps = []
        for c in range(nch):
            cps.append(pltpu.async_copy(
                table_hbm.at[idx_v.at[pl.ds(c * _CH, _CH)]],
                bufs[c % 2], sems[c % 2]))
            if c >= 1:
                cps[c - 1].wait()
                pltpu.sync_copy(bufs[(c - 1) % 2],
                                out_hbm.at[pl.ds(base + (c - 1) * _CH, _CH)])
        cps[nch - 1].wait()
        pltpu.sync_copy(bufs[(nch - 1) % 2],
                        out_hbm.at[pl.ds(base + (nch - 1) * _CH, _CH)])

    return k(table, idx)


def kernel(predicted_positions, real_positions, real_expressions):
    B, N, _ = predicted_positions.shape
    G = real_expressions.shape[2]
    qs = (predicted_positions * -2.0).astype(jnp.bfloat16)
    kt = real_positions.transpose(0, 2, 1)
    ks = kt.astype(jnp.bfloat16)
    idx = _nearest_indices(predicted_positions, qs, ks, kt)
    table = real_expressions.reshape(B * N, G)
    out = _gather_rows_sc(table, idx.reshape(B * N))
    return out.reshape(B, N, G)


# speedup vs baseline: 3.6532x; 1.1766x over previous
import functools

import jax
import jax.numpy as jnp
from jax import lax
from jax.experimental import pallas as pl
from jax.experimental.pallas import tpu as pltpu
from jax.experimental.pallas import tpu_sc as plsc

_QB = 512
_BIG = 2 ** 30


def _argmin_body(n_keys, q_ref, qs_ref, ks_ref, kt_ref, out_ref):
    q = q_ref[0]
    a2 = jnp.sum(q * q, axis=1, keepdims=True)
    k = kt_ref[0]
    b2 = jnp.sum(k * k, axis=0, keepdims=True)
    mm = jnp.dot(qs_ref[0], ks_ref[0], preferred_element_type=jnp.float32)
    d2 = (a2 + b2) + mm
    m = jnp.min(d2, axis=1, keepdims=True)
    mc = jnp.maximum(m, 0.0)
    iota = lax.broadcasted_iota(jnp.int32, (1, d2.shape[1]), 1).astype(jnp.float32)
    idx = jnp.min(jnp.where(d2 <= mc, iota, jnp.float32(_BIG)),
                  axis=1, keepdims=True)
    out_ref[0] = idx.astype(jnp.int32) + pl.program_id(0) * n_keys


def _nearest_indices(pred, qs, ks, kt):
    B, N, _ = pred.shape
    grid = (B, N // _QB)
    return pl.pallas_call(
        functools.partial(_argmin_body, N),
        grid=grid,
        in_specs=[
            pl.BlockSpec((1, _QB, 2), lambda b, i: (b, i, 0)),
            pl.BlockSpec((1, _QB, 2), lambda b, i: (b, i, 0)),
            pl.BlockSpec((1, 2, N), lambda b, i: (b, 0, 0)),
            pl.BlockSpec((1, 2, N), lambda b, i: (b, 0, 0)),
        ],
        out_specs=pl.BlockSpec((1, _QB, 1), lambda b, i: (b * (N // _QB) + i, 0, 0)),
        out_shape=jax.ShapeDtypeStruct((B * (N // _QB), _QB, 1), jnp.int32),
    )(pred, qs, ks, kt)


_NC = 2
_NS = 16
_NW = _NC * _NS
_CH = 128


def _gather_rows_sc(table, idx):
    rows, G = table.shape
    per_w = rows // _NW
    nch = per_w // _CH
    mesh = plsc.VectorSubcoreMesh(core_axis_name="c", subcore_axis_name="s")

    @functools.partial(
        pl.kernel, mesh=mesh,
        out_type=jax.ShapeDtypeStruct((rows, G), jnp.float32),
        scratch_types=[
            pltpu.VMEM((per_w,), jnp.int32),
            pltpu.VMEM((_CH, G), jnp.float32),
            pltpu.VMEM((_CH, G), jnp.float32),
            pltpu.SemaphoreType.DMA,
            pltpu.SemaphoreType.DMA,
        ],
    )
    def k(table_hbm, idx_hbm, out_hbm, idx_v, buf0, buf1, sem0, sem1):
        wid = lax.axis_index("s") * _NC + lax.axis_index("c")
        base = wid * per_w
        pltpu.sync_copy(idx_hbm.at[pl.ds(base, per_w)], idx_v)
        bufs = (buf0, buf1)
        sems = (sem0, sem1)
        cps = []
        for c in range(nch):
            cps.append(pltpu.async_copy(
                table_hbm.at[idx_v.at[pl.ds(c * _CH, _CH)]],
                bufs[c % 2], sems[c % 2]))
            if c >= 1:
                cps[c - 1].wait()
                pltpu.sync_copy(bufs[(c - 1) % 2],
                                out_hbm.at[pl.ds(base + (c - 1) * _CH, _CH)])
        cps[nch - 1].wait()
        pltpu.sync_copy(bufs[(nch - 1) % 2],
                        out_hbm.at[pl.ds(base + (nch - 1) * _CH, _CH)])

    return k(table, idx)


def kernel(predicted_positions, real_positions, real_expressions):
    B, N, _ = predicted_positions.shape
    G = real_expressions.shape[2]
    qs = (predicted_positions * -2.0).astype(jnp.bfloat16)
    kt = real_positions.transpose(0, 2, 1)
    ks = kt.astype(jnp.bfloat16)
    idx = _nearest_indices(predicted_positions, qs, ks, kt)
    table = real_expressions.reshape(B * N, G)
    out = _gather_rows_sc(table, idx.reshape(B * N))
    return out.reshape(B, N, G)


# speedup vs baseline: 3.7536x; 1.0275x over previous
import functools

import jax
import jax.numpy as jnp
from jax import lax
from jax.experimental import pallas as pl
from jax.experimental.pallas import tpu as pltpu
from jax.experimental.pallas import tpu_sc as plsc

_QB = 1024
_BIG = 2 ** 30


def _argmin_body(n_keys, q_ref, qs_ref, ks_ref, kt_ref, out_ref):
    q = q_ref[0]
    a2 = jnp.sum(q * q, axis=1, keepdims=True)
    k = kt_ref[0]
    b2 = jnp.sum(k * k, axis=0, keepdims=True)
    mm = jnp.dot(qs_ref[0], ks_ref[0], preferred_element_type=jnp.float32)
    m = jnp.min((a2 + b2) + mm, axis=1, keepdims=True)
    mc = jnp.maximum(m, 0.0)
    iota = lax.broadcasted_iota(jnp.int32, (1, mm.shape[1]), 1).astype(jnp.float32)
    idx = jnp.min(jnp.where(((a2 + b2) + mm) <= mc, iota, jnp.float32(_BIG)),
                  axis=1, keepdims=True)
    out_ref[0] = idx.astype(jnp.int32) + pl.program_id(0) * n_keys


def _nearest_indices(pred, qs, ks, kt):
    B, N, _ = pred.shape
    grid = (B, N // _QB)
    return pl.pallas_call(
        functools.partial(_argmin_body, N),
        grid=grid,
        in_specs=[
            pl.BlockSpec((1, _QB, 2), lambda b, i: (b, i, 0)),
            pl.BlockSpec((1, _QB, 2), lambda b, i: (b, i, 0)),
            pl.BlockSpec((1, 2, N), lambda b, i: (b, 0, 0)),
            pl.BlockSpec((1, 2, N), lambda b, i: (b, 0, 0)),
        ],
        out_specs=pl.BlockSpec((1, _QB, 1), lambda b, i: (b * (N // _QB) + i, 0, 0)),
        out_shape=jax.ShapeDtypeStruct((B * (N // _QB), _QB, 1), jnp.int32),
    )(pred, qs, ks, kt)


_NC = 2
_NS = 16
_NW = _NC * _NS
_CH = 128


def _gather_rows_sc(table, idx):
    rows, G = table.shape
    per_w = rows // _NW
    nch = per_w // _CH
    mesh = plsc.VectorSubcoreMesh(core_axis_name="c", subcore_axis_name="s")

    @functools.partial(
        pl.kernel, mesh=mesh,
        out_type=jax.ShapeDtypeStruct((rows, G), jnp.float32),
        scratch_types=[
            pltpu.VMEM((per_w,), jnp.int32),
            pltpu.VMEM((_CH, G), jnp.float32),
            pltpu.VMEM((_CH, G), jnp.float32),
            pltpu.SemaphoreType.DMA,
            pltpu.SemaphoreType.DMA,
        ],
    )
    def k(table_hbm, idx_hbm, out_hbm, idx_v, buf0, buf1, sem0, sem1):
        wid = lax.axis_index("s") * _NC + lax.axis_index("c")
        base = wid * per_w
        pltpu.sync_copy(idx_hbm.at[pl.ds(base, per_w)], idx_v)
        bufs = (buf0, buf1)
        sems = (sem0, sem1)
        cps = []
        for c in range(nch):
            cps.append(pltpu.async_copy(
                table_hbm.at[idx_v.at[pl.ds(c * _CH, _CH)]],
                bufs[c % 2], sems[c % 2]))
            if c >= 1:
                cps[c - 1].wait()
                pltpu.sync_copy(bufs[(c - 1) % 2],
                                out_hbm.at[pl.ds(base + (c - 1) * _CH, _CH)])
        cps[nch - 1].wait()
        pltpu.sync_copy(bufs[(nch - 1) % 2],
                        out_hbm.at[pl.ds(base + (nch - 1) * _CH, _CH)])

    return k(table, idx)


def kernel(predicted_positions, real_positions, real_expressions):
    B, N, _ = predicted_positions.shape
    G = real_expressions.shape[2]
    qs = (predicted_positions * -2.0).astype(jnp.bfloat16)
    kt = real_positions.transpose(0, 2, 1)
    ks = kt.astype(jnp.bfloat16)
    idx = _nearest_indices(predicted_positions, qs, ks, kt)
    table = real_expressions.reshape(B * N, G)
    out = _gather_rows_sc(table, idx.reshape(B * N))
    return out.reshape(B, N, G)
